# baseline (speedup 1.0000x reference)
import numpy as np
import jax
import jax.numpy as jnp
from jax.experimental import pallas as pl

D_MODEL = 1024
RANK = 128
N_INPUT = 8
N_PROCESS = 32
N_OUTPUT = 8
K = 4

TILE = 512

def _body(x_ref, wi_ref, idx_ref, wo_ref, win_ref, pn_ref, won_ref, out_ref):
    T = x_ref.shape[0]
    f32 = jnp.float32
    bf16 = jnp.bfloat16

    P = jnp.dot(x_ref[...].astype(bf16), win_ref[...], preferred_element_type=f32)
    wi = wi_ref[...]
    h = P[:, :RANK] * wi[:, 0:1]
    for n in range(1, N_INPUT):
        h = h + P[:, n * RANK:(n + 1) * RANK] * wi[:, n : n + 1]

    pn = pn_ref[...]
    pnhat = pn * jax.lax.rsqrt(jnp.sum(pn * pn, axis=1, keepdims=True) + 1e-8)
    pnhat2_16 = (pnhat * 2.0).astype(bf16)
    pnhatT16 = pnhat.T.astype(bf16)
    idx = idx_ref[...]
    iota = jax.lax.broadcasted_iota(jnp.int32, (T, N_PROCESS), 1)
    for i in range(K):
        g = jnp.dot(h.astype(bf16), pnhatT16, preferred_element_type=f32)
        oh = idx[:, i : i + 1] == iota
        sel = jnp.where(oh, g, 0.0)
        h = h - jnp.dot(sel.astype(bf16), pnhat2_16, preferred_element_type=f32)

    h16 = h.astype(bf16)
    wo16 = wo_ref[...].astype(bf16)
    hw = jnp.concatenate(
        [h16 * wo16[:, n : n + 1] for n in range(N_OUTPUT)], axis=1)
    out_ref[...] = jnp.dot(hw, won_ref[...], preferred_element_type=f32)


def kernel(x, input_weights, process_indices, output_weights,
           input_neurons, process_neurons, output_neurons):
    B, S, D = x.shape
    N = B * S
    xf = x.reshape(N, D)
    wif = input_weights.reshape(N, N_INPUT)
    idxf = process_indices.reshape(N, K).astype(jnp.int32)
    wof = output_weights.reshape(N, N_OUTPUT)
    win16 = jnp.transpose(input_neurons, (1, 0, 2)).reshape(D, N_INPUT * RANK).astype(jnp.bfloat16)
    won16 = output_neurons.reshape(N_OUTPUT * RANK, D).astype(jnp.bfloat16)

    grid = (N // TILE,)
    out = pl.pallas_call(
        _body,
        grid=grid,
        in_specs=[
            pl.BlockSpec((TILE, D), lambda i: (i, 0)),
            pl.BlockSpec((TILE, N_INPUT), lambda i: (i, 0)),
            pl.BlockSpec((TILE, K), lambda i: (i, 0)),
            pl.BlockSpec((TILE, N_OUTPUT), lambda i: (i, 0)),
            pl.BlockSpec((D, N_INPUT * RANK), lambda i: (0, 0)),
            pl.BlockSpec((N_PROCESS, RANK), lambda i: (0, 0)),
            pl.BlockSpec((N_OUTPUT * RANK, D), lambda i: (0, 0)),
        ],
        out_specs=pl.BlockSpec((TILE, D), lambda i: (i, 0)),
        out_shape=jax.ShapeDtypeStruct((N, D), jnp.float32),
    )(xf, wif, idxf, wof, win16, process_neurons, won16)
    return out.reshape(B, S, D)


# speedup vs baseline: 1.1525x; 1.1525x over previous
import numpy as np
import jax
import jax.numpy as jnp
from jax.experimental import pallas as pl

D_MODEL = 1024
RANK = 128
N_INPUT = 8
N_PROCESS = 32
N_OUTPUT = 8
K = 4

TILE = 2048

def _body(x_ref, wi_ref, idx_ref, wo_ref, win_ref, pn_ref, won_ref, out_ref):
    T = x_ref.shape[0]
    f32 = jnp.float32
    bf16 = jnp.bfloat16

    P = jnp.dot(x_ref[...].astype(bf16), win_ref[...], preferred_element_type=f32)
    wi = wi_ref[...]
    h = P[:, :RANK] * wi[:, 0:1]
    for n in range(1, N_INPUT):
        h = h + P[:, n * RANK:(n + 1) * RANK] * wi[:, n : n + 1]

    pn = pn_ref[...]
    pnhat = pn * jax.lax.rsqrt(jnp.sum(pn * pn, axis=1, keepdims=True) + 1e-8)
    pnhat2_16 = (pnhat * 2.0).astype(bf16)
    pnhatT16 = pnhat.T.astype(bf16)
    idx = idx_ref[...]
    iota = jax.lax.broadcasted_iota(jnp.int32, (T, N_PROCESS), 1)
    for i in range(K):
        g = jnp.dot(h.astype(bf16), pnhatT16, preferred_element_type=f32)
        oh = idx[:, i : i + 1] == iota
        sel = jnp.where(oh, g, 0.0)
        h = h - jnp.dot(sel.astype(bf16), pnhat2_16, preferred_element_type=f32)

    h16 = h.astype(bf16)
    wo16 = wo_ref[...].astype(bf16)
    hw = jnp.concatenate(
        [h16 * wo16[:, n : n + 1] for n in range(N_OUTPUT)], axis=1)
    out_ref[...] = jnp.dot(hw, won_ref[...], preferred_element_type=f32)


def kernel(x, input_weights, process_indices, output_weights,
           input_neurons, process_neurons, output_neurons):
    B, S, D = x.shape
    N = B * S
    xf = x.reshape(N, D)
    wif = input_weights.reshape(N, N_INPUT)
    idxf = process_indices.reshape(N, K).astype(jnp.int32)
    wof = output_weights.reshape(N, N_OUTPUT)
    win16 = jnp.transpose(input_neurons, (1, 0, 2)).reshape(D, N_INPUT * RANK).astype(jnp.bfloat16)
    won16 = output_neurons.reshape(N_OUTPUT * RANK, D).astype(jnp.bfloat16)

    grid = (N // TILE,)
    out = pl.pallas_call(
        _body,
        grid=grid,
        in_specs=[
            pl.BlockSpec((TILE, D), lambda i: (i, 0)),
            pl.BlockSpec((TILE, N_INPUT), lambda i: (i, 0)),
            pl.BlockSpec((TILE, K), lambda i: (i, 0)),
            pl.BlockSpec((TILE, N_OUTPUT), lambda i: (i, 0)),
            pl.BlockSpec((D, N_INPUT * RANK), lambda i: (0, 0)),
            pl.BlockSpec((N_PROCESS, RANK), lambda i: (0, 0)),
            pl.BlockSpec((N_OUTPUT * RANK, D), lambda i: (0, 0)),
        ],
        out_specs=pl.BlockSpec((TILE, D), lambda i: (i, 0)),
        out_shape=jax.ShapeDtypeStruct((N, D), jnp.float32),
    )(xf, wif, idxf, wof, win16, process_neurons, won16)
    return out.reshape(B, S, D)


# speedup vs baseline: 1.1719x; 1.0168x over previous
import numpy as np
import jax
import jax.numpy as jnp
from jax.experimental import pallas as pl
from jax.experimental.pallas import tpu as pltpu

D_MODEL = 1024
RANK = 128
N_INPUT = 8
N_PROCESS = 32
N_OUTPUT = 8
K = 4

TILE = 1024

def _body(x_ref, wi_ref, idx_ref, wo_ref, win_ref, pn_ref, won_ref, out_ref,
          win16_s, won16_s):
    T = x_ref.shape[0]
    f32 = jnp.float32
    bf16 = jnp.bfloat16

    @pl.when(pl.program_id(0) == 0)
    def _prep():
        for n in range(N_INPUT):
            win16_s[:, n * RANK:(n + 1) * RANK] = win_ref[n].astype(bf16)
        won16_s[...] = won_ref[...].astype(bf16)

    P = jnp.dot(x_ref[...].astype(bf16), win16_s[...], preferred_element_type=f32)
    wi = wi_ref[...]
    h = P[:, :RANK] * wi[:, 0:1]
    for n in range(1, N_INPUT):
        h = h + P[:, n * RANK:(n + 1) * RANK] * wi[:, n : n + 1]

    pn = pn_ref[...]
    pnhat = pn * jax.lax.rsqrt(jnp.sum(pn * pn, axis=1, keepdims=True) + 1e-8)
    pnhat2_16 = (pnhat * 2.0).astype(bf16)
    pnhatT16 = pnhat.T.astype(bf16)
    idx = idx_ref[...]
    iota = jax.lax.broadcasted_iota(jnp.int32, (T, N_PROCESS), 1)
    for i in range(K):
        g = jnp.dot(h.astype(bf16), pnhatT16, preferred_element_type=f32)
        oh = idx[:, i : i + 1] == iota
        sel = jnp.where(oh, g, 0.0)
        h = h - jnp.dot(sel.astype(bf16), pnhat2_16, preferred_element_type=f32)

    h16 = h.astype(bf16)
    wo16 = wo_ref[...].astype(bf16)
    hw = jnp.concatenate(
        [h16 * wo16[:, n : n + 1] for n in range(N_OUTPUT)], axis=1)
    out_ref[...] = jnp.dot(hw, won16_s[...], preferred_element_type=f32)


def kernel(x, input_weights, process_indices, output_weights,
           input_neurons, process_neurons, output_neurons):
    B, S, D = x.shape
    N = B * S
    xf = x.reshape(N, D)
    wif = input_weights.reshape(N, N_INPUT)
    idxf = process_indices.reshape(N, K).astype(jnp.int32)
    wof = output_weights.reshape(N, N_OUTPUT)
    won_flat = output_neurons.reshape(N_OUTPUT * RANK, D)

    grid = (N // TILE,)
    out = pl.pallas_call(
        _body,
        grid=grid,
        in_specs=[
            pl.BlockSpec((TILE, D), lambda i: (i, 0)),
            pl.BlockSpec((TILE, N_INPUT), lambda i: (i, 0)),
            pl.BlockSpec((TILE, K), lambda i: (i, 0)),
            pl.BlockSpec((TILE, N_OUTPUT), lambda i: (i, 0)),
            pl.BlockSpec((N_INPUT, D, RANK), lambda i: (0, 0, 0)),
            pl.BlockSpec((N_PROCESS, RANK), lambda i: (0, 0)),
            pl.BlockSpec((N_OUTPUT * RANK, D), lambda i: (0, 0)),
        ],
        out_specs=pl.BlockSpec((TILE, D), lambda i: (i, 0)),
        out_shape=jax.ShapeDtypeStruct((N, D), jnp.float32),
        scratch_shapes=[
            pltpu.VMEM((D, N_INPUT * RANK), jnp.bfloat16),
            pltpu.VMEM((N_OUTPUT * RANK, D), jnp.bfloat16),
        ],
    )(xf, wif, idxf, wof, input_neurons, process_neurons, won_flat)
    return out.reshape(B, S, D)


# speedup vs baseline: 1.2219x; 1.0427x over previous
import numpy as np
import jax
import jax.numpy as jnp
from jax.experimental import pallas as pl
from jax.experimental.pallas import tpu as pltpu

D_MODEL = 1024
RANK = 128
N_INPUT = 8
N_PROCESS = 32
N_OUTPUT = 8
K = 4

TILE = 1024

def _body(x_ref, wi_ref, idx_ref, wo_ref, win_ref, pn_ref, won_ref, out_ref,
          win16_s, won16_s):
    T = x_ref.shape[0]
    f32 = jnp.float32
    bf16 = jnp.bfloat16

    @pl.when(pl.program_id(0) == 0)
    def _prep():
        for n in range(N_INPUT):
            win16_s[:, n * RANK:(n + 1) * RANK] = win_ref[n].astype(bf16)
        won16_s[...] = won_ref[...].astype(bf16)

    P = jnp.dot(x_ref[...].astype(bf16), win16_s[...], preferred_element_type=f32)
    wi = wi_ref[...]
    h = P[:, :RANK] * wi[:, 0:1]
    for n in range(1, N_INPUT):
        h = h + P[:, n * RANK:(n + 1) * RANK] * wi[:, n : n + 1]

    pn = pn_ref[...]
    pnhat = pn * jax.lax.rsqrt(jnp.sum(pn * pn, axis=1, keepdims=True) + 1e-8)
    pnhat16 = pnhat.astype(bf16)
    pnhatT16 = pnhat.T.astype(bf16)
    c16 = jnp.dot(pnhat16, pnhatT16, preferred_element_type=f32).astype(bf16)
    idx = idx_ref[...]
    iota = jax.lax.broadcasted_iota(jnp.int32, (T, N_PROCESS), 1)
    g0 = jnp.dot(h.astype(bf16), pnhatT16, preferred_element_type=f32)
    A = 2.0 * jnp.where(idx[:, 0:1] == iota, g0, 0.0)
    for i in range(1, K):
        corr = jnp.dot(A.astype(bf16), c16, preferred_element_type=f32)
        A = A + 2.0 * jnp.where(idx[:, i : i + 1] == iota, g0 - corr, 0.0)
    h = h - jnp.dot(A.astype(bf16), pnhat16, preferred_element_type=f32)

    h16 = h.astype(bf16)
    wo16 = wo_ref[...].astype(bf16)
    hw = jnp.concatenate(
        [h16 * wo16[:, n : n + 1] for n in range(N_OUTPUT)], axis=1)
    out_ref[...] = jnp.dot(hw, won16_s[...], preferred_element_type=f32)


def kernel(x, input_weights, process_indices, output_weights,
           input_neurons, process_neurons, output_neurons):
    B, S, D = x.shape
    N = B * S
    xf = x.reshape(N, D)
    wif = input_weights.reshape(N, N_INPUT)
    idxf = process_indices.reshape(N, K).astype(jnp.int32)
    wof = output_weights.reshape(N, N_OUTPUT)
    won_flat = output_neurons.reshape(N_OUTPUT * RANK, D)

    grid = (N // TILE,)
    out = pl.pallas_call(
        _body,
        grid=grid,
        in_specs=[
            pl.BlockSpec((TILE, D), lambda i: (i, 0)),
            pl.BlockSpec((TILE, N_INPUT), lambda i: (i, 0)),
            pl.BlockSpec((TILE, K), lambda i: (i, 0)),
            pl.BlockSpec((TILE, N_OUTPUT), lambda i: (i, 0)),
            pl.BlockSpec((N_INPUT, D, RANK), lambda i: (0, 0, 0)),
            pl.BlockSpec((N_PROCESS, RANK), lambda i: (0, 0)),
            pl.BlockSpec((N_OUTPUT * RANK, D), lambda i: (0, 0)),
        ],
        out_specs=pl.BlockSpec((TILE, D), lambda i: (i, 0)),
        out_shape=jax.ShapeDtypeStruct((N, D), jnp.float32),
        scratch_shapes=[
            pltpu.VMEM((D, N_INPUT * RANK), jnp.bfloat16),
            pltpu.VMEM((N_OUTPUT * RANK, D), jnp.bfloat16),
        ],
    )(xf, wif, idxf, wof, input_neurons, process_neurons, won_flat)
    return out.reshape(B, S, D)
